# Initial kernel scaffold; baseline (speedup 1.0000x reference)
import functools

import jax
import jax.numpy as jnp
from jax import lax
from jax.experimental import pallas as pl
from jax.experimental.pallas import tpu as pltpu
from jax.experimental.pallas import tpu_sc as plsc

N = 10000
E = 320000
D = 128
NC = 2
NS = 16
NW = NC * NS
EPT = E // NW
CE = 80
NCHUNK = EPT // CE
RPT = N // NS
LANES = 16


def _mm_body(x_ref, w_ref, o_ref):
    o_ref[...] = jnp.dot(x_ref[...], w_ref[...],
                         preferred_element_type=jnp.float32)


def _matmul(X, W):
    bm = 1000
    return pl.pallas_call(
        _mm_body,
        grid=(N // bm,),
        in_specs=[
            pl.BlockSpec((bm, D), lambda i: (i, 0)),
            pl.BlockSpec((D, D), lambda i: (0, 0)),
        ],
        out_specs=pl.BlockSpec((bm, D), lambda i: (i, 0)),
        out_shape=jax.ShapeDtypeStruct((N, D), jnp.float32),
    )(X, W)


def _spmm_body(hw_hbm, rows_hbm, cols_hbm, vals_hbm, part_hbm,
               acc, rbuf, cbuf, vbuf, gbuf, sem):
    c = lax.axis_index("c")
    s = lax.axis_index("s")
    wid = c * NS + s

    zeros16 = jnp.zeros((LANES,), jnp.float32)

    def zrow(e, carry):
        for f in range(D // LANES):
            gbuf[e, pl.ds(LANES * f, LANES)] = zeros16
        return carry

    lax.fori_loop(0, CE, zrow, 0)

    zbase = s * RPT
    for j in range(RPT // CE):
        pltpu.sync_copy(gbuf, acc.at[pl.ds(zbase + j * CE, CE)])
    rem = RPT - (RPT // CE) * CE
    pltpu.sync_copy(gbuf.at[pl.ds(0, rem)],
                    acc.at[pl.ds(zbase + (RPT // CE) * CE, rem)])

    plsc.subcore_barrier()

    ebase = wid * EPT

    def chunk_body(i, carry):
        base = ebase + i * CE
        pltpu.sync_copy(rows_hbm.at[pl.ds(base, CE)], rbuf)
        pltpu.sync_copy(cols_hbm.at[pl.ds(base, CE)], cbuf)
        pltpu.sync_copy(vals_hbm.at[pl.ds(base, CE)], vbuf)
        pltpu.async_copy(hw_hbm.at[cbuf], gbuf, sem).wait()

        def erow(e, c2):
            v = plsc.load_gather(vbuf, [lax.broadcast(e, (LANES,))])
            for f in range(D // LANES):
                sl = pl.ds(LANES * f, LANES)
                gbuf[e, sl] = gbuf[e, sl] * v
            return c2

        lax.fori_loop(0, CE, erow, 0)

        pltpu.sync_copy(gbuf, acc.at[rbuf], add=True)
        return carry

    lax.fori_loop(0, NCHUNK, chunk_body, 0)

    plsc.subcore_barrier()

    obase = s * RPT
    pltpu.sync_copy(acc.at[pl.ds(obase, RPT)],
                    part_hbm.at[c, pl.ds(obase, RPT)])


def _spmm(HW, rows, cols, vals):
    mesh = plsc.VectorSubcoreMesh(core_axis_name="c", subcore_axis_name="s")
    f = functools.partial(
        pl.kernel,
        out_type=jax.ShapeDtypeStruct((NC, N, D), jnp.float32),
        mesh=mesh,
        scratch_types=[
            pltpu.VMEM_SHARED((N, D), jnp.float32),
            pltpu.VMEM((CE,), jnp.int32),
            pltpu.VMEM((CE,), jnp.int32),
            pltpu.VMEM((CE,), jnp.float32),
            pltpu.VMEM((CE, D), jnp.float32),
            pltpu.SemaphoreType.DMA,
        ],
    )(_spmm_body)
    return f(HW, rows, cols, vals)


def _comb_body(p0_ref, p1_ref, b_ref, o_ref):
    o_ref[...] = p0_ref[...] + p1_ref[...] + b_ref[...]


def _combine(p0, p1, b2d):
    bm = 1000
    return pl.pallas_call(
        _comb_body,
        grid=(N // bm,),
        in_specs=[
            pl.BlockSpec((bm, D), lambda i: (i, 0)),
            pl.BlockSpec((bm, D), lambda i: (i, 0)),
            pl.BlockSpec((1, D), lambda i: (0, 0)),
        ],
        out_specs=pl.BlockSpec((bm, D), lambda i: (i, 0)),
        out_shape=jax.ShapeDtypeStruct((N, D), jnp.float32),
    )(p0, p1, b2d)


def kernel(X, adj_indices, adj_values, W, b, mediators):
    HW = _matmul(X, W)
    rows = adj_indices[0]
    cols = adj_indices[1]
    part = _spmm(HW, rows, cols, adj_values)
    return _combine(part[0], part[1], b.reshape(1, D))


# speedup vs baseline: 3.6296x; 3.6296x over previous
import functools

import jax
import jax.numpy as jnp
from jax import lax
from jax.experimental import pallas as pl
from jax.experimental.pallas import tpu as pltpu
from jax.experimental.pallas import tpu_sc as plsc

N = 10000
E = 320000
D = 128
NC = 2
NS = 16
NW = NC * NS
EPT = E // NW
CE = 80
NCHUNK = EPT // CE
RPT = 624
RPT_LAST = N - RPT * (NS - 1)
LANES = 16


def _mm_body(x_ref, w_ref, o_ref):
    o_ref[...] = jnp.dot(x_ref[...], w_ref[...],
                         preferred_element_type=jnp.float32)


def _matmul(X, W):
    bm = 1000
    return pl.pallas_call(
        _mm_body,
        grid=(N // bm,),
        in_specs=[
            pl.BlockSpec((bm, D), lambda i: (i, 0)),
            pl.BlockSpec((D, D), lambda i: (0, 0)),
        ],
        out_specs=pl.BlockSpec((bm, D), lambda i: (i, 0)),
        out_shape=jax.ShapeDtypeStruct((N, D), jnp.float32),
    )(X, W)


def _spmm_body(hw_hbm, rows_hbm, cols_hbm, vals_hbm, part_hbm,
               acc, rbuf, cbuf, vbuf, gbuf, sem):
    c = lax.axis_index("c")
    s = lax.axis_index("s")
    wid = c * NS + s

    zeros16 = jnp.zeros((LANES,), jnp.float32)

    def zrow(e, carry):
        for f in range(D // LANES):
            gbuf[e, pl.ds(LANES * f, LANES)] = zeros16
        return carry

    lax.fori_loop(0, CE, zrow, 0)

    def copy_zero(r0, nr):
        nfull = nr // CE
        for j in range(nfull):
            pltpu.sync_copy(gbuf, acc.at[pl.ds(r0 + j * CE, CE)])
        rem = nr - nfull * CE
        if rem:
            pltpu.sync_copy(gbuf.at[pl.ds(0, rem)],
                            acc.at[pl.ds(r0 + nfull * CE, rem)])

    @pl.when(s < NS - 1)
    def _():
        copy_zero(s * RPT, RPT)

    @pl.when(s == NS - 1)
    def _():
        copy_zero((NS - 1) * RPT, RPT_LAST)

    plsc.subcore_barrier()

    ebase = wid * EPT

    def chunk_body(i, carry):
        base = ebase + i * CE
        pltpu.sync_copy(rows_hbm.at[pl.ds(base, CE)], rbuf)
        pltpu.sync_copy(cols_hbm.at[pl.ds(base, CE)], cbuf)
        pltpu.sync_copy(vals_hbm.at[pl.ds(base, CE)], vbuf)
        pltpu.async_copy(hw_hbm.at[cbuf], gbuf, sem).wait()

        def erow(e, c2):
            v = plsc.load_gather(vbuf, [lax.broadcast(e, (LANES,))])
            for f in range(D // LANES):
                sl = pl.ds(LANES * f, LANES)
                gbuf[e, sl] = gbuf[e, sl] * v
            return c2

        lax.fori_loop(0, CE, erow, 0)

        pltpu.sync_copy(gbuf, acc.at[rbuf], add=True)
        return carry

    lax.fori_loop(0, NCHUNK, chunk_body, 0)

    plsc.subcore_barrier()

    @pl.when(s < NS - 1)
    def _():
        obase = s * RPT
        pltpu.sync_copy(acc.at[pl.ds(obase, RPT)],
                        part_hbm.at[c, pl.ds(obase, RPT)])

    @pl.when(s == NS - 1)
    def _():
        obase = (NS - 1) * RPT
        pltpu.sync_copy(acc.at[pl.ds(obase, RPT_LAST)],
                        part_hbm.at[c, pl.ds(obase, RPT_LAST)])


def _spmm(HW, rows, cols, vals):
    mesh = plsc.VectorSubcoreMesh(core_axis_name="c", subcore_axis_name="s")
    f = functools.partial(
        pl.kernel,
        out_type=jax.ShapeDtypeStruct((NC, N, D), jnp.float32),
        mesh=mesh,
        compiler_params=pltpu.CompilerParams(needs_layout_passes=False),
        scratch_types=[
            pltpu.VMEM_SHARED((N, D), jnp.float32),
            pltpu.VMEM((CE,), jnp.int32),
            pltpu.VMEM((CE,), jnp.int32),
            pltpu.VMEM((CE,), jnp.float32),
            pltpu.VMEM((CE, D), jnp.float32),
            pltpu.SemaphoreType.DMA,
        ],
    )(_spmm_body)
    return f(HW, rows, cols, vals)


def _comb_body(p0_ref, p1_ref, b_ref, o_ref):
    o_ref[...] = p0_ref[...] + p1_ref[...] + b_ref[...]


def _combine(p0, p1, b2d):
    bm = 1000
    return pl.pallas_call(
        _comb_body,
        grid=(N // bm,),
        in_specs=[
            pl.BlockSpec((bm, D), lambda i: (i, 0)),
            pl.BlockSpec((bm, D), lambda i: (i, 0)),
            pl.BlockSpec((1, D), lambda i: (0, 0)),
        ],
        out_specs=pl.BlockSpec((bm, D), lambda i: (i, 0)),
        out_shape=jax.ShapeDtypeStruct((N, D), jnp.float32),
    )(p0, p1, b2d)


def kernel(X, adj_indices, adj_values, W, b, mediators):
    HW = _matmul(X, W)
    rows = adj_indices[0]
    cols = adj_indices[1]
    part = _spmm(HW, rows, cols, adj_values)
    return _combine(part[0], part[1], b.reshape(1, D))


# speedup vs baseline: 7.2043x; 1.9849x over previous
import functools

import jax
import jax.numpy as jnp
from jax import lax
from jax.experimental import pallas as pl
from jax.experimental.pallas import tpu as pltpu
from jax.experimental.pallas import tpu_sc as plsc

N = 10000
E = 320000
D = 128
NC = 2
NS = 16
NW = NC * NS
EPT = E // NW
CE = 80
NCHUNK = EPT // CE
RPT = 624
RPT_LAST = N - RPT * (NS - 1)
LANES = 16


def _mm_body(x_ref, w_ref, o_ref):
    o_ref[...] = jnp.dot(x_ref[...], w_ref[...],
                         preferred_element_type=jnp.float32)


def _matmul(X, W):
    bm = 1000
    return pl.pallas_call(
        _mm_body,
        grid=(N // bm,),
        in_specs=[
            pl.BlockSpec((bm, D), lambda i: (i, 0)),
            pl.BlockSpec((D, D), lambda i: (0, 0)),
        ],
        out_specs=pl.BlockSpec((bm, D), lambda i: (i, 0)),
        out_shape=jax.ShapeDtypeStruct((N, D), jnp.float32),
    )(X, W)


NB = 4
NG = NCHUNK // NB
TAIL_OFF = NG * NB * CE


def _spmm_body(hw_hbm, rows_hbm, cols_hbm, vals_hbm, part_hbm,
               acc, *bufs):
    cbufs = bufs[0:NB]
    vbufs = bufs[NB:2 * NB]
    rbufs = bufs[2 * NB:3 * NB]
    gbufs = bufs[3 * NB:4 * NB]
    csems = bufs[4 * NB:5 * NB]
    vsems = bufs[5 * NB:6 * NB]
    rsems = bufs[6 * NB:7 * NB]
    gsems = bufs[7 * NB:8 * NB]
    ssems = bufs[8 * NB:9 * NB]

    c = lax.axis_index("c")
    s = lax.axis_index("s")
    wid = c * NS + s
    ebase = wid * EPT

    def pf_cols(b, off):
        off = jnp.minimum(off, EPT - CE)
        pltpu.async_copy(cols_hbm.at[pl.ds(ebase + off, CE)],
                         cbufs[b], csems[b])

    def pf_vals(b, off):
        off = jnp.minimum(off, EPT - CE)
        pltpu.async_copy(vals_hbm.at[pl.ds(ebase + off, CE)],
                         vbufs[b], vsems[b])

    def pf_rows(b, off):
        off = jnp.minimum(off, EPT - CE)
        pltpu.async_copy(rows_hbm.at[pl.ds(ebase + off, CE)],
                         rbufs[b], rsems[b])

    def wait_cols(b):
        pltpu.make_async_copy(cols_hbm.at[pl.ds(ebase, CE)],
                              cbufs[b], csems[b]).wait()

    def wait_vals(b):
        pltpu.make_async_copy(vals_hbm.at[pl.ds(ebase, CE)],
                              vbufs[b], vsems[b]).wait()

    def wait_rows(b):
        pltpu.make_async_copy(rows_hbm.at[pl.ds(ebase, CE)],
                              rbufs[b], rsems[b]).wait()

    for b in range(NB):
        pf_cols(b, b * CE)
        pf_vals(b, b * CE)
        pf_rows(b, b * CE)

    zeros16 = jnp.zeros((LANES,), jnp.float32)

    def zrow(e, carry):
        for f in range(D // LANES):
            gbufs[0][e, pl.ds(LANES * f, LANES)] = zeros16
        return carry

    lax.fori_loop(0, CE, zrow, 0)

    def copy_zero(r0, nr):
        nfull = nr // CE
        for j in range(nfull):
            pltpu.sync_copy(gbufs[0], acc.at[pl.ds(r0 + j * CE, CE)])
        rem = nr - nfull * CE
        if rem:
            pltpu.sync_copy(gbufs[0].at[pl.ds(0, rem)],
                            acc.at[pl.ds(r0 + nfull * CE, rem)])

    @pl.when(s < NS - 1)
    def _():
        copy_zero(s * RPT, RPT)

    @pl.when(s == NS - 1)
    def _():
        copy_zero((NS - 1) * RPT, RPT_LAST)

    plsc.subcore_barrier()

    def scale_rows(buf, vbuf):
        def erow(e, c2):
            v = plsc.load_gather(vbuf, [lax.broadcast(e, (LANES,))])
            for f in range(D // LANES):
                sl = pl.ds(LANES * f, LANES)
                buf[e, sl] = buf[e, sl] * v
            return c2

        lax.fori_loop(0, CE, erow, 0)

    def group(g, carry):
        gbase = g * NB * CE
        nbase = gbase + NB * CE
        gds = []
        for b in range(NB):
            wait_cols(b)
            gds.append(pltpu.async_copy(hw_hbm.at[cbufs[b]],
                                        gbufs[b], gsems[b]))
        sds = []
        for b in range(NB):
            gds[b].wait()
            pf_cols(b, nbase + b * CE)
            wait_vals(b)
            scale_rows(gbufs[b], vbufs[b])
            pf_vals(b, nbase + b * CE)
            wait_rows(b)
            sds.append(pltpu.async_copy(gbufs[b], acc.at[rbufs[b]],
                                        ssems[b], add=True))
        for b in range(NB):
            sds[b].wait()
            pf_rows(b, nbase + b * CE)
        return carry

    lax.fori_loop(0, NG, group, 0)

    wait_cols(0)
    gd = pltpu.async_copy(hw_hbm.at[cbufs[0]], gbufs[0], gsems[0])
    gd.wait()
    wait_vals(0)
    scale_rows(gbufs[0], vbufs[0])
    wait_rows(0)
    pltpu.async_copy(gbufs[0], acc.at[rbufs[0]], ssems[0], add=True).wait()
    for b in range(1, NB):
        wait_cols(b)
        wait_vals(b)
        wait_rows(b)

    plsc.subcore_barrier()

    @pl.when(s < NS - 1)
    def _():
        obase = s * RPT
        pltpu.sync_copy(acc.at[pl.ds(obase, RPT)],
                        part_hbm.at[c, pl.ds(obase, RPT)])

    @pl.when(s == NS - 1)
    def _():
        obase = (NS - 1) * RPT
        pltpu.sync_copy(acc.at[pl.ds(obase, RPT_LAST)],
                        part_hbm.at[c, pl.ds(obase, RPT_LAST)])


def _spmm(HW, rows, cols, vals):
    mesh = plsc.VectorSubcoreMesh(core_axis_name="c", subcore_axis_name="s")
    f = functools.partial(
        pl.kernel,
        out_type=jax.ShapeDtypeStruct((NC, N, D), jnp.float32),
        mesh=mesh,
        compiler_params=pltpu.CompilerParams(needs_layout_passes=False),
        scratch_types=(
            [pltpu.VMEM_SHARED((N, D), jnp.float32)]
            + [pltpu.VMEM((CE,), jnp.int32)] * NB
            + [pltpu.VMEM((CE,), jnp.float32)] * NB
            + [pltpu.VMEM((CE,), jnp.int32)] * NB
            + [pltpu.VMEM((CE, D), jnp.float32)] * NB
            + [pltpu.SemaphoreType.DMA] * (5 * NB)
        ),
    )(_spmm_body)
    return f(HW, rows, cols, vals)


def _comb_body(p0_ref, p1_ref, b_ref, o_ref):
    o_ref[...] = p0_ref[...] + p1_ref[...] + b_ref[...]


def _combine(p0, p1, b2d):
    bm = 1000
    return pl.pallas_call(
        _comb_body,
        grid=(N // bm,),
        in_specs=[
            pl.BlockSpec((bm, D), lambda i: (i, 0)),
            pl.BlockSpec((bm, D), lambda i: (i, 0)),
            pl.BlockSpec((1, D), lambda i: (0, 0)),
        ],
        out_specs=pl.BlockSpec((bm, D), lambda i: (i, 0)),
        out_shape=jax.ShapeDtypeStruct((N, D), jnp.float32),
    )(p0, p1, b2d)


def kernel(X, adj_indices, adj_values, W, b, mediators):
    HW = _matmul(X, W)
    rows = adj_indices[0]
    cols = adj_indices[1]
    part = _spmm(HW, rows, cols, adj_values)
    return _combine(part[0], part[1], b.reshape(1, D))


# speedup vs baseline: 8.4690x; 1.1755x over previous
import functools

import jax
import jax.numpy as jnp
from jax import lax
from jax.experimental import pallas as pl
from jax.experimental.pallas import tpu as pltpu
from jax.experimental.pallas import tpu_sc as plsc

N = 10000
E = 320000
D = 128
NC = 2
NS = 16
NW = NC * NS
EPT = E // NW
CE = 80
NCHUNK = EPT // CE
RPT = 624
RPT_LAST = N - RPT * (NS - 1)
LANES = 16


def _mm_body(x_ref, w_ref, o_ref):
    o_ref[...] = jnp.dot(x_ref[...], w_ref[...],
                         preferred_element_type=jnp.float32)


def _matmul(X, W):
    bm = 1000
    return pl.pallas_call(
        _mm_body,
        grid=(N // bm,),
        in_specs=[
            pl.BlockSpec((bm, D), lambda i: (i, 0)),
            pl.BlockSpec((D, D), lambda i: (0, 0)),
        ],
        out_specs=pl.BlockSpec((bm, D), lambda i: (i, 0)),
        out_shape=jax.ShapeDtypeStruct((N, D), jnp.float32),
    )(X, W)


NB = 4
NG = NCHUNK // NB
TAIL_OFF = NG * NB * CE


def _spmm_body(hw_hbm, rows_hbm, cols_hbm, vals_hbm, part_hbm,
               acc, *bufs):
    cbufs = bufs[0:NB]
    vbufs = bufs[NB:2 * NB]
    rbufs = bufs[2 * NB:3 * NB]
    gbufs = bufs[3 * NB:4 * NB]
    csems = bufs[4 * NB:5 * NB]
    vsems = bufs[5 * NB:6 * NB]
    rsems = bufs[6 * NB:7 * NB]
    gsems = bufs[7 * NB:8 * NB]
    ssems = bufs[8 * NB:9 * NB]

    c = lax.axis_index("c")
    s = lax.axis_index("s")
    wid = c * NS + s
    ebase = wid * EPT

    def pf_cols(b, off):
        off = jnp.minimum(off, EPT - CE)
        pltpu.async_copy(cols_hbm.at[pl.ds(ebase + off, CE)],
                         cbufs[b], csems[b])

    def pf_vals(b, off):
        off = jnp.minimum(off, EPT - CE)
        pltpu.async_copy(vals_hbm.at[pl.ds(ebase + off, CE)],
                         vbufs[b], vsems[b])

    def pf_rows(b, off):
        off = jnp.minimum(off, EPT - CE)
        pltpu.async_copy(rows_hbm.at[pl.ds(ebase + off, CE)],
                         rbufs[b], rsems[b])

    def wait_cols(b):
        pltpu.make_async_copy(cols_hbm.at[pl.ds(ebase, CE)],
                              cbufs[b], csems[b]).wait()

    def wait_vals(b):
        pltpu.make_async_copy(vals_hbm.at[pl.ds(ebase, CE)],
                              vbufs[b], vsems[b]).wait()

    def wait_rows(b):
        pltpu.make_async_copy(rows_hbm.at[pl.ds(ebase, CE)],
                              rbufs[b], rsems[b]).wait()

    for b in range(NB):
        pf_cols(b, b * CE)
        pf_vals(b, b * CE)
        pf_rows(b, b * CE)

    zeros16 = jnp.zeros((LANES,), jnp.float32)

    @plsc.parallel_loop(0, CE, step=1, unroll=4)
    def _(e):
        for f in range(D // LANES):
            gbufs[0][e, pl.ds(LANES * f, LANES)] = zeros16

    def copy_zero(r0, nr):
        nfull = nr // CE
        for j in range(nfull):
            pltpu.sync_copy(gbufs[0], acc.at[pl.ds(r0 + j * CE, CE)])
        rem = nr - nfull * CE
        if rem:
            pltpu.sync_copy(gbufs[0].at[pl.ds(0, rem)],
                            acc.at[pl.ds(r0 + nfull * CE, rem)])

    @pl.when(s < NS - 1)
    def _():
        copy_zero(s * RPT, RPT)

    @pl.when(s == NS - 1)
    def _():
        copy_zero((NS - 1) * RPT, RPT_LAST)

    plsc.subcore_barrier()

    def scale_rows(buf, vbuf):
        @plsc.parallel_loop(0, CE, step=1, unroll=4)
        def _(e):
            v = plsc.load_gather(vbuf, [lax.broadcast(e, (LANES,))])
            for f in range(D // LANES):
                sl = pl.ds(LANES * f, LANES)
                buf[e, sl] = buf[e, sl] * v

    def group(g, carry):
        gbase = g * NB * CE
        nbase = gbase + NB * CE
        gds = []
        for b in range(NB):
            wait_cols(b)
            gds.append(pltpu.async_copy(hw_hbm.at[cbufs[b]],
                                        gbufs[b], gsems[b]))
        sds = []
        for b in range(NB):
            gds[b].wait()
            pf_cols(b, nbase + b * CE)
            wait_vals(b)
            scale_rows(gbufs[b], vbufs[b])
            pf_vals(b, nbase + b * CE)
            wait_rows(b)
            sds.append(pltpu.async_copy(gbufs[b], acc.at[rbufs[b]],
                                        ssems[b], add=True))
        for b in range(NB):
            sds[b].wait()
            pf_rows(b, nbase + b * CE)
        return carry

    lax.fori_loop(0, NG, group, 0)

    wait_cols(0)
    gd = pltpu.async_copy(hw_hbm.at[cbufs[0]], gbufs[0], gsems[0])
    gd.wait()
    wait_vals(0)
    scale_rows(gbufs[0], vbufs[0])
    wait_rows(0)
    pltpu.async_copy(gbufs[0], acc.at[rbufs[0]], ssems[0], add=True).wait()
    for b in range(1, NB):
        wait_cols(b)
        wait_vals(b)
        wait_rows(b)

    plsc.subcore_barrier()

    @pl.when(s < NS - 1)
    def _():
        obase = s * RPT
        pltpu.sync_copy(acc.at[pl.ds(obase, RPT)],
                        part_hbm.at[c, pl.ds(obase, RPT)])

    @pl.when(s == NS - 1)
    def _():
        obase = (NS - 1) * RPT
        pltpu.sync_copy(acc.at[pl.ds(obase, RPT_LAST)],
                        part_hbm.at[c, pl.ds(obase, RPT_LAST)])


def _spmm(HW, rows, cols, vals):
    mesh = plsc.VectorSubcoreMesh(core_axis_name="c", subcore_axis_name="s")
    f = functools.partial(
        pl.kernel,
        out_type=jax.ShapeDtypeStruct((NC, N, D), jnp.float32),
        mesh=mesh,
        compiler_params=pltpu.CompilerParams(needs_layout_passes=False),
        scratch_types=(
            [pltpu.VMEM_SHARED((N, D), jnp.float32)]
            + [pltpu.VMEM((CE,), jnp.int32)] * NB
            + [pltpu.VMEM((CE,), jnp.float32)] * NB
            + [pltpu.VMEM((CE,), jnp.int32)] * NB
            + [pltpu.VMEM((CE, D), jnp.float32)] * NB
            + [pltpu.SemaphoreType.DMA] * (5 * NB)
        ),
    )(_spmm_body)
    return f(HW, rows, cols, vals)


def _comb_body(p0_ref, p1_ref, b_ref, o_ref):
    o_ref[...] = p0_ref[...] + p1_ref[...] + b_ref[...]


def _combine(p0, p1, b2d):
    bm = 1000
    return pl.pallas_call(
        _comb_body,
        grid=(N // bm,),
        in_specs=[
            pl.BlockSpec((bm, D), lambda i: (i, 0)),
            pl.BlockSpec((bm, D), lambda i: (i, 0)),
            pl.BlockSpec((1, D), lambda i: (0, 0)),
        ],
        out_specs=pl.BlockSpec((bm, D), lambda i: (i, 0)),
        out_shape=jax.ShapeDtypeStruct((N, D), jnp.float32),
    )(p0, p1, b2d)


def kernel(X, adj_indices, adj_values, W, b, mediators):
    HW = _matmul(X, W)
    rows = adj_indices[0]
    cols = adj_indices[1]
    part = _spmm(HW, rows, cols, adj_values)
    return _combine(part[0], part[1], b.reshape(1, D))


# speedup vs baseline: 9.8036x; 1.1576x over previous
import functools

import jax
import jax.numpy as jnp
from jax import lax
from jax.experimental import pallas as pl
from jax.experimental.pallas import tpu as pltpu
from jax.experimental.pallas import tpu_sc as plsc

N = 10000
E = 320000
D = 128
NC = 2
NS = 16
NW = NC * NS
EPT = E // NW
CE = 80
NCHUNK = EPT // CE
RPT = 624
RPT_LAST = N - RPT * (NS - 1)
LANES = 16


def _mm_body(x_ref, w_ref, o_ref):
    o_ref[...] = jnp.dot(x_ref[...], w_ref[...],
                         preferred_element_type=jnp.float32)


def _matmul(X, W):
    bm = 1000
    return pl.pallas_call(
        _mm_body,
        grid=(N // bm,),
        in_specs=[
            pl.BlockSpec((bm, D), lambda i: (i, 0)),
            pl.BlockSpec((D, D), lambda i: (0, 0)),
        ],
        out_specs=pl.BlockSpec((bm, D), lambda i: (i, 0)),
        out_shape=jax.ShapeDtypeStruct((N, D), jnp.float32),
    )(X, W)


NB = 4
NG = NCHUNK // NB
TAIL_OFF = NG * NB * CE


def _spmm_body(hw_hbm, rows_hbm, cols_hbm, vals_hbm, part_hbm,
               acc, *bufs):
    cbufs = bufs[0:NB]
    vbufs = bufs[NB:2 * NB]
    rbufs = bufs[2 * NB:3 * NB]
    gbufs = bufs[3 * NB:4 * NB]
    csems = bufs[4 * NB:5 * NB]
    vsems = bufs[5 * NB:6 * NB]
    rsems = bufs[6 * NB:7 * NB]
    gsems = bufs[7 * NB:8 * NB]
    ssems = bufs[8 * NB:9 * NB]

    c = lax.axis_index("c")
    s = lax.axis_index("s")
    wid = c * NS + s
    ebase = wid * EPT

    def pf_cols(b, off):
        off = jnp.minimum(off, EPT - CE)
        pltpu.async_copy(cols_hbm.at[pl.ds(ebase + off, CE)],
                         cbufs[b], csems[b])

    def pf_vals(b, off):
        off = jnp.minimum(off, EPT - CE)
        pltpu.async_copy(vals_hbm.at[pl.ds(ebase + off, CE)],
                         vbufs[b], vsems[b])

    def pf_rows(b, off):
        off = jnp.minimum(off, EPT - CE)
        pltpu.async_copy(rows_hbm.at[pl.ds(ebase + off, CE)],
                         rbufs[b], rsems[b])

    def wait_cols(b):
        pltpu.make_async_copy(cols_hbm.at[pl.ds(ebase, CE)],
                              cbufs[b], csems[b]).wait()

    def wait_vals(b):
        pltpu.make_async_copy(vals_hbm.at[pl.ds(ebase, CE)],
                              vbufs[b], vsems[b]).wait()

    def wait_rows(b):
        pltpu.make_async_copy(rows_hbm.at[pl.ds(ebase, CE)],
                              rbufs[b], rsems[b]).wait()

    for b in range(NB):
        pf_cols(b, b * CE)
        pf_vals(b, b * CE)
        pf_rows(b, b * CE)

    zeros16 = jnp.zeros((LANES,), jnp.float32)

    @plsc.parallel_loop(0, CE, step=1, unroll=4)
    def _(e):
        for f in range(D // LANES):
            gbufs[0][e, pl.ds(LANES * f, LANES)] = zeros16

    def copy_zero(r0, nr):
        nfull = nr // CE
        for j in range(nfull):
            pltpu.sync_copy(gbufs[0], acc.at[pl.ds(r0 + j * CE, CE)])
        rem = nr - nfull * CE
        if rem:
            pltpu.sync_copy(gbufs[0].at[pl.ds(0, rem)],
                            acc.at[pl.ds(r0 + nfull * CE, rem)])

    @pl.when(s < NS - 1)
    def _():
        copy_zero(s * RPT, RPT)

    @pl.when(s == NS - 1)
    def _():
        copy_zero((NS - 1) * RPT, RPT_LAST)

    plsc.subcore_barrier()

    def scale_rows(buf, vbuf):
        @plsc.parallel_loop(0, CE, step=1, unroll=8)
        def _(e):
            v = plsc.load_gather(vbuf, [lax.broadcast(e, (LANES,))])
            for f in range(D // LANES):
                sl = pl.ds(LANES * f, LANES)
                buf[e, sl] = buf[e, sl] * v

    def wait_scatter(b):
        pltpu.make_async_copy(gbufs[b], acc.at[rbufs[b]], ssems[b]).wait()

    def group(g, carry):
        gbase = g * NB * CE
        nbase = gbase + NB * CE
        gds = []
        for b in range(NB):
            @pl.when(g > 0)
            def _(b=b, off=gbase + b * CE):
                wait_scatter(b)
                pf_rows(b, off)

            wait_cols(b)
            gds.append(pltpu.async_copy(hw_hbm.at[cbufs[b]],
                                        gbufs[b], gsems[b]))
        for b in range(NB):
            gds[b].wait()
            pf_cols(b, nbase + b * CE)
            wait_vals(b)
            scale_rows(gbufs[b], vbufs[b])
            pf_vals(b, nbase + b * CE)
            wait_rows(b)
            pltpu.async_copy(gbufs[b], acc.at[rbufs[b]], ssems[b],
                             add=True)
        return carry

    lax.fori_loop(0, NG, group, 0)

    wait_scatter(0)
    pf_rows(0, TAIL_OFF)
    wait_cols(0)
    gd = pltpu.async_copy(hw_hbm.at[cbufs[0]], gbufs[0], gsems[0])
    gd.wait()
    wait_vals(0)
    scale_rows(gbufs[0], vbufs[0])
    wait_rows(0)
    pltpu.async_copy(gbufs[0], acc.at[rbufs[0]], ssems[0], add=True).wait()
    for b in range(1, NB):
        wait_scatter(b)
        wait_cols(b)
        wait_vals(b)

    plsc.subcore_barrier()

    @pl.when(s < NS - 1)
    def _():
        obase = s * RPT
        pltpu.sync_copy(acc.at[pl.ds(obase, RPT)],
                        part_hbm.at[c, pl.ds(obase, RPT)])

    @pl.when(s == NS - 1)
    def _():
        obase = (NS - 1) * RPT
        pltpu.sync_copy(acc.at[pl.ds(obase, RPT_LAST)],
                        part_hbm.at[c, pl.ds(obase, RPT_LAST)])


def _spmm(HW, rows, cols, vals):
    mesh = plsc.VectorSubcoreMesh(core_axis_name="c", subcore_axis_name="s")
    f = functools.partial(
        pl.kernel,
        out_type=jax.ShapeDtypeStruct((NC, N, D), jnp.float32),
        mesh=mesh,
        compiler_params=pltpu.CompilerParams(needs_layout_passes=False),
        scratch_types=(
            [pltpu.VMEM_SHARED((N, D), jnp.float32)]
            + [pltpu.VMEM((CE,), jnp.int32)] * NB
            + [pltpu.VMEM((CE,), jnp.float32)] * NB
            + [pltpu.VMEM((CE,), jnp.int32)] * NB
            + [pltpu.VMEM((CE, D), jnp.float32)] * NB
            + [pltpu.SemaphoreType.DMA] * (5 * NB)
        ),
    )(_spmm_body)
    return f(HW, rows, cols, vals)


def _comb_body(p0_ref, p1_ref, b_ref, o_ref):
    o_ref[...] = p0_ref[...] + p1_ref[...] + b_ref[...]


def _combine(p0, p1, b2d):
    bm = 1000
    return pl.pallas_call(
        _comb_body,
        grid=(N // bm,),
        in_specs=[
            pl.BlockSpec((bm, D), lambda i: (i, 0)),
            pl.BlockSpec((bm, D), lambda i: (i, 0)),
            pl.BlockSpec((1, D), lambda i: (0, 0)),
        ],
        out_specs=pl.BlockSpec((bm, D), lambda i: (i, 0)),
        out_shape=jax.ShapeDtypeStruct((N, D), jnp.float32),
    )(p0, p1, b2d)


def kernel(X, adj_indices, adj_values, W, b, mediators):
    HW = _matmul(X, W)
    rows = adj_indices[0]
    cols = adj_indices[1]
    part = _spmm(HW, rows, cols, adj_values)
    return _combine(part[0], part[1], b.reshape(1, D))


# speedup vs baseline: 10.4063x; 1.0615x over previous
import functools

import jax
import jax.numpy as jnp
from jax import lax
from jax.experimental import pallas as pl
from jax.experimental.pallas import tpu as pltpu
from jax.experimental.pallas import tpu_sc as plsc

N = 10000
E = 320000
D = 128
NC = 2
NS = 16
NW = NC * NS
EPT = E // NW
CE = 80
NCHUNK = EPT // CE
RPT = 624
RPT_LAST = N - RPT * (NS - 1)
LANES = 16
NB = 4
NG = NCHUNK // NB
TAIL_OFF = NG * NB * CE


def _mm_body(x_ref, w_ref, o_ref):
    o_ref[...] = jnp.dot(x_ref[...], w_ref[...],
                         preferred_element_type=jnp.float32)


def _matmul(X, W):
    bm = 2000
    return pl.pallas_call(
        _mm_body,
        grid=(N // bm,),
        in_specs=[
            pl.BlockSpec((bm, D), lambda i: (i, 0)),
            pl.BlockSpec((D, D), lambda i: (0, 0)),
        ],
        out_specs=pl.BlockSpec((bm, D), lambda i: (i, 0)),
        out_shape=jax.ShapeDtypeStruct((N, D), jnp.float32),
    )(X, W)


def _spmm_body(hw_hbm, rows_hbm, cols_hbm, vals_hbm, part0_hbm, part1_hbm,
               acc, *bufs):
    cbufs = bufs[0:NB]
    vbufs = bufs[NB:2 * NB]
    rbufs = bufs[2 * NB:3 * NB]
    gbufs = bufs[3 * NB:4 * NB]
    csems = bufs[4 * NB:5 * NB]
    vsems = bufs[5 * NB:6 * NB]
    rsems = bufs[6 * NB:7 * NB]
    gsems = bufs[7 * NB:8 * NB]
    ssems = bufs[8 * NB:9 * NB]

    c = lax.axis_index("c")
    s = lax.axis_index("s")
    wid = c * NS + s
    ebase = wid * EPT

    def pf_cols(b, off):
        off = jnp.minimum(off, EPT - CE)
        pltpu.async_copy(cols_hbm.at[pl.ds(ebase + off, CE)],
                         cbufs[b], csems[b])

    def pf_vals(b, off):
        off = jnp.minimum(off, EPT - CE)
        pltpu.async_copy(vals_hbm.at[pl.ds(ebase + off, CE)],
                         vbufs[b], vsems[b])

    def pf_rows(b, off):
        off = jnp.minimum(off, EPT - CE)
        pltpu.async_copy(rows_hbm.at[pl.ds(ebase + off, CE)],
                         rbufs[b], rsems[b])

    def wait_cols(b):
        pltpu.make_async_copy(cols_hbm.at[pl.ds(ebase, CE)],
                              cbufs[b], csems[b]).wait()

    def wait_vals(b):
        pltpu.make_async_copy(vals_hbm.at[pl.ds(ebase, CE)],
                              vbufs[b], vsems[b]).wait()

    def wait_rows(b):
        pltpu.make_async_copy(rows_hbm.at[pl.ds(ebase, CE)],
                              rbufs[b], rsems[b]).wait()

    for b in range(NB):
        pf_cols(b, b * CE)
        pf_vals(b, b * CE)
        pf_rows(b, b * CE)

    zeros16 = jnp.zeros((LANES,), jnp.float32)

    @plsc.parallel_loop(0, CE, step=1, unroll=4)
    def _(e):
        for f in range(D // LANES):
            gbufs[0][e, pl.ds(LANES * f, LANES)] = zeros16

    def copy_zero(r0, nr):
        nfull = nr // CE
        for j in range(nfull):
            pltpu.sync_copy(gbufs[0], acc.at[pl.ds(r0 + j * CE, CE)])
        rem = nr - nfull * CE
        if rem:
            pltpu.sync_copy(gbufs[0].at[pl.ds(0, rem)],
                            acc.at[pl.ds(r0 + nfull * CE, rem)])

    @pl.when(s < NS - 1)
    def _():
        copy_zero(s * RPT, RPT)

    @pl.when(s == NS - 1)
    def _():
        copy_zero((NS - 1) * RPT, RPT_LAST)

    plsc.subcore_barrier()

    def scale_rows(buf, vbuf):
        @plsc.parallel_loop(0, CE, step=1, unroll=8)
        def _(e):
            v = plsc.load_gather(vbuf, [lax.broadcast(e, (LANES,))])
            for f in range(D // LANES):
                sl = pl.ds(LANES * f, LANES)
                buf[e, sl] = buf[e, sl] * v

    def wait_scatter(b):
        pltpu.make_async_copy(gbufs[b], acc.at[rbufs[b]], ssems[b]).wait()

    def group(g, carry):
        gbase = g * NB * CE
        nbase = gbase + NB * CE
        gds = []
        for b in range(NB):
            @pl.when(g > 0)
            def _(b=b, off=gbase + b * CE):
                wait_scatter(b)
                pf_rows(b, off)

            wait_cols(b)
            gds.append(pltpu.async_copy(hw_hbm.at[cbufs[b]],
                                        gbufs[b], gsems[b]))
        for b in range(NB):
            gds[b].wait()
            pf_cols(b, nbase + b * CE)
            wait_vals(b)
            scale_rows(gbufs[b], vbufs[b])
            pf_vals(b, nbase + b * CE)
            wait_rows(b)
            pltpu.async_copy(gbufs[b], acc.at[rbufs[b]], ssems[b],
                             add=True)
        return carry

    lax.fori_loop(0, NG, group, 0)

    wait_scatter(0)
    pf_rows(0, TAIL_OFF)
    wait_cols(0)
    gd = pltpu.async_copy(hw_hbm.at[cbufs[0]], gbufs[0], gsems[0])
    gd.wait()
    wait_vals(0)
    scale_rows(gbufs[0], vbufs[0])
    wait_rows(0)
    pltpu.async_copy(gbufs[0], acc.at[rbufs[0]], ssems[0], add=True).wait()
    for b in range(1, NB):
        wait_scatter(b)
        wait_cols(b)
        wait_vals(b)

    plsc.subcore_barrier()

    def write_out(r0, nr):
        @pl.when(c == 0)
        def _():
            pltpu.sync_copy(acc.at[pl.ds(r0, nr)],
                            part0_hbm.at[pl.ds(r0, nr)])

        @pl.when(c == 1)
        def _():
            pltpu.sync_copy(acc.at[pl.ds(r0, nr)],
                            part1_hbm.at[pl.ds(r0, nr)])

    @pl.when(s < NS - 1)
    def _():
        write_out(s * RPT, RPT)

    @pl.when(s == NS - 1)
    def _():
        write_out((NS - 1) * RPT, RPT_LAST)


def _spmm(HW, rows, cols, vals):
    mesh = plsc.VectorSubcoreMesh(core_axis_name="c", subcore_axis_name="s")
    f = functools.partial(
        pl.kernel,
        out_type=(jax.ShapeDtypeStruct((N, D), jnp.float32),
                  jax.ShapeDtypeStruct((N, D), jnp.float32)),
        mesh=mesh,
        compiler_params=pltpu.CompilerParams(needs_layout_passes=False),
        scratch_types=(
            [pltpu.VMEM_SHARED((N, D), jnp.float32)]
            + [pltpu.VMEM((CE,), jnp.int32)] * NB
            + [pltpu.VMEM((CE,), jnp.float32)] * NB
            + [pltpu.VMEM((CE,), jnp.int32)] * NB
            + [pltpu.VMEM((CE, D), jnp.float32)] * NB
            + [pltpu.SemaphoreType.DMA] * (5 * NB)
        ),
    )(_spmm_body)
    return f(HW, rows, cols, vals)


def _comb_body(p0_ref, p1_ref, b_ref, o_ref):
    o_ref[...] = p0_ref[...] + p1_ref[...] + b_ref[...]


def _combine(p0, p1, b2d):
    bm = 2000
    return pl.pallas_call(
        _comb_body,
        grid=(N // bm,),
        in_specs=[
            pl.BlockSpec((bm, D), lambda i: (i, 0)),
            pl.BlockSpec((bm, D), lambda i: (i, 0)),
            pl.BlockSpec((1, D), lambda i: (0, 0)),
        ],
        out_specs=pl.BlockSpec((bm, D), lambda i: (i, 0)),
        out_shape=jax.ShapeDtypeStruct((N, D), jnp.float32),
    )(p0, p1, b2d)


def kernel(X, adj_indices, adj_values, W, b, mediators):
    HW = _matmul(X, W)
    p0, p1 = _spmm(HW, adj_indices[0], adj_indices[1], adj_values)
    return _combine(p0, p1, b.reshape(1, D))


# speedup vs baseline: 11.3485x; 1.0905x over previous
import functools

import jax
import jax.numpy as jnp
from jax import lax
from jax.experimental import pallas as pl
from jax.experimental.pallas import tpu as pltpu
from jax.experimental.pallas import tpu_sc as plsc

N = 10000
E = 320000
D = 128
NC = 2
NS = 16
NW = NC * NS
CE = 128
NCHUNK = E // CE
KFULL = NCHUNK // NW
NEXTRA = NCHUNK - KFULL * NW
RPT = 624
RPT_LAST = N - RPT * (NS - 1)
LANES = 16
NB = 3
NG = KFULL // NB


def _mm_body(x_ref, w_ref, o_ref):
    o_ref[...] = jnp.dot(x_ref[...], w_ref[...],
                         preferred_element_type=jnp.float32)


def _matmul(X, W):
    bm = 2000
    return pl.pallas_call(
        _mm_body,
        grid=(N // bm,),
        in_specs=[
            pl.BlockSpec((bm, D), lambda i: (i, 0)),
            pl.BlockSpec((D, D), lambda i: (0, 0)),
        ],
        out_specs=pl.BlockSpec((bm, D), lambda i: (i, 0)),
        out_shape=jax.ShapeDtypeStruct((N, D), jnp.float32),
    )(X, W)


def _spmm_body(hw_hbm, adj_hbm, vals_hbm, part0_hbm, part1_hbm,
               acc, *bufs):
    rcbufs = bufs[0:NB]
    vbufs = bufs[NB:2 * NB]
    rbufs = bufs[2 * NB:3 * NB]
    gbufs = bufs[3 * NB:4 * NB]
    rcsems = bufs[4 * NB:5 * NB]
    vsems = bufs[5 * NB:6 * NB]
    gsems = bufs[6 * NB:7 * NB]
    ssems = bufs[7 * NB:8 * NB]

    c = lax.axis_index("c")
    s = lax.axis_index("s")
    wid = c * NS + s

    def off_of(k):
        return jnp.minimum((k * NW + wid) * CE, E - CE)

    def pf_rc(b, k):
        pltpu.async_copy(adj_hbm.at[pl.ds(0, 2), pl.ds(off_of(k), CE)],
                         rcbufs[b], rcsems[b])

    def pf_vals(b, k):
        pltpu.async_copy(vals_hbm.at[pl.ds(off_of(k), CE)],
                         vbufs[b], vsems[b])

    def wait_rc(b):
        pltpu.make_async_copy(adj_hbm.at[pl.ds(0, 2), pl.ds(0, CE)],
                              rcbufs[b], rcsems[b]).wait()

    def wait_vals(b):
        pltpu.make_async_copy(vals_hbm.at[pl.ds(0, CE)],
                              vbufs[b], vsems[b]).wait()

    def copy_rows(b):
        for j in range(CE // LANES):
            sl = pl.ds(LANES * j, LANES)
            rbufs[b][sl] = rcbufs[b][0, sl]

    for b in range(NB):
        pf_rc(b, b)
        pf_vals(b, b)

    zeros16 = jnp.zeros((LANES,), jnp.float32)

    @plsc.parallel_loop(0, CE, step=1, unroll=4)
    def _(e):
        for f in range(D // LANES):
            gbufs[0][e, pl.ds(LANES * f, LANES)] = zeros16

    def copy_zero(r0, nr):
        nfull = nr // CE
        for j in range(nfull):
            pltpu.sync_copy(gbufs[0], acc.at[pl.ds(r0 + j * CE, CE)])
        rem = nr - nfull * CE
        if rem:
            pltpu.sync_copy(gbufs[0].at[pl.ds(0, rem)],
                            acc.at[pl.ds(r0 + nfull * CE, rem)])

    @pl.when(s < NS - 1)
    def _():
        copy_zero(s * RPT, RPT)

    @pl.when(s == NS - 1)
    def _():
        copy_zero((NS - 1) * RPT, RPT_LAST)

    plsc.subcore_barrier()

    def scale_rows(buf, vbuf):
        @plsc.parallel_loop(0, CE, step=1, unroll=8)
        def _(e):
            v = plsc.load_gather(vbuf, [lax.broadcast(e, (LANES,))])
            for f in range(D // LANES):
                sl = pl.ds(LANES * f, LANES)
                buf[e, sl] = buf[e, sl] * v

    def wait_scatter(b):
        pltpu.make_async_copy(gbufs[b], acc.at[rbufs[b]], ssems[b]).wait()

    def group(g, carry):
        kbase = g * NB
        gds = []
        for b in range(NB):
            @pl.when(g > 0)
            def _(b=b):
                wait_scatter(b)

            wait_rc(b)
            copy_rows(b)
            gds.append(pltpu.async_copy(hw_hbm.at[rcbufs[b].at[1]],
                                        gbufs[b], gsems[b]))
        for b in range(NB):
            gds[b].wait()
            pf_rc(b, kbase + NB + b)
            wait_vals(b)
            scale_rows(gbufs[b], vbufs[b])
            pf_vals(b, kbase + NB + b)
            pltpu.async_copy(gbufs[b], acc.at[rbufs[b]], ssems[b],
                             add=True)
        return carry

    lax.fori_loop(0, NG, group, 0)

    @pl.when(wid < NEXTRA)
    def _():
        wait_scatter(0)
        wait_rc(0)
        copy_rows(0)
        pltpu.async_copy(hw_hbm.at[rcbufs[0].at[1]], gbufs[0],
                         gsems[0]).wait()
        wait_vals(0)
        scale_rows(gbufs[0], vbufs[0])
        pltpu.async_copy(gbufs[0], acc.at[rbufs[0]], ssems[0],
                         add=True).wait()
        for b in range(1, NB):
            wait_scatter(b)
            wait_rc(b)
            wait_vals(b)

    @pl.when(wid >= NEXTRA)
    def _():
        for b in range(NB):
            wait_scatter(b)
            wait_rc(b)
            wait_vals(b)

    plsc.subcore_barrier()

    def write_out(r0, nr):
        @pl.when(c == 0)
        def _():
            pltpu.sync_copy(acc.at[pl.ds(r0, nr)],
                            part0_hbm.at[pl.ds(r0, nr)])

        @pl.when(c == 1)
        def _():
            pltpu.sync_copy(acc.at[pl.ds(r0, nr)],
                            part1_hbm.at[pl.ds(r0, nr)])

    @pl.when(s < NS - 1)
    def _():
        write_out(s * RPT, RPT)

    @pl.when(s == NS - 1)
    def _():
        write_out((NS - 1) * RPT, RPT_LAST)


def _spmm(HW, adj_indices, vals):
    mesh = plsc.VectorSubcoreMesh(core_axis_name="c", subcore_axis_name="s")
    f = functools.partial(
        pl.kernel,
        out_type=(jax.ShapeDtypeStruct((N, D), jnp.float32),
                  jax.ShapeDtypeStruct((N, D), jnp.float32)),
        mesh=mesh,
        compiler_params=pltpu.CompilerParams(needs_layout_passes=False),
        scratch_types=(
            [pltpu.VMEM_SHARED((N, D), jnp.float32)]
            + [pltpu.VMEM((2, CE), jnp.int32)] * NB
            + [pltpu.VMEM((CE,), jnp.float32)] * NB
            + [pltpu.VMEM((CE,), jnp.int32)] * NB
            + [pltpu.VMEM((CE, D), jnp.float32)] * NB
            + [pltpu.SemaphoreType.DMA] * (4 * NB)
        ),
    )(_spmm_body)
    return f(HW, adj_indices, vals)


def _comb_body(p0_ref, p1_ref, b_ref, o_ref):
    o_ref[...] = p0_ref[...] + p1_ref[...] + b_ref[...]


def _combine(p0, p1, b2d):
    bm = 2000
    return pl.pallas_call(
        _comb_body,
        grid=(N // bm,),
        in_specs=[
            pl.BlockSpec((bm, D), lambda i: (i, 0)),
            pl.BlockSpec((bm, D), lambda i: (i, 0)),
            pl.BlockSpec((1, D), lambda i: (0, 0)),
        ],
        out_specs=pl.BlockSpec((bm, D), lambda i: (i, 0)),
        out_shape=jax.ShapeDtypeStruct((N, D), jnp.float32),
    )(p0, p1, b2d)


def kernel(X, adj_indices, adj_values, W, b, mediators):
    HW = _matmul(X, W)
    p0, p1 = _spmm(HW, adj_indices, adj_values)
    return _combine(p0, p1, b.reshape(1, D))


# speedup vs baseline: 11.7946x; 1.0393x over previous
import functools

import jax
import jax.numpy as jnp
from jax import lax
from jax.experimental import pallas as pl
from jax.experimental.pallas import tpu as pltpu
from jax.experimental.pallas import tpu_sc as plsc

N = 10000
E = 320000
D = 128
NC = 2
NS = 16
NW = NC * NS
CE = 128
NCHUNK = E // CE
KFULL = NCHUNK // NW
NEXTRA = NCHUNK - KFULL * NW
RPT = 624
RPT_LAST = N - RPT * (NS - 1)
LANES = 16
SUB = 64
NR = 3
NG = KFULL // NR
NXT = [(1, 1), (2, 0), (2, 1), (0, 0), (0, 1), (1, 0)]


def _mm_body(x_ref, w_ref, o_ref):
    o_ref[...] = jnp.dot(x_ref[...], w_ref[...],
                         preferred_element_type=jnp.float32)


def _matmul(X, W):
    bm = 2000
    return pl.pallas_call(
        _mm_body,
        grid=(N // bm,),
        in_specs=[
            pl.BlockSpec((bm, D), lambda i: (i, 0)),
            pl.BlockSpec((D, D), lambda i: (0, 0)),
        ],
        out_specs=pl.BlockSpec((bm, D), lambda i: (i, 0)),
        out_shape=jax.ShapeDtypeStruct((N, D), jnp.float32),
    )(X, W)


def _spmm_body(hw_hbm, adj_hbm, vals_hbm, part0_hbm, part1_hbm,
               acc, *bufs):
    rcbufs = bufs[0:NR]
    vbufs = bufs[NR:2 * NR]
    rbufs = bufs[2 * NR:3 * NR]
    gbufs = bufs[3 * NR:4 * NR]
    sbufs = bufs[4 * NR:5 * NR]
    rcsems = bufs[5 * NR:6 * NR]
    vsems = bufs[6 * NR:7 * NR]
    gsems = bufs[7 * NR:8 * NR]
    ssems = bufs[8 * NR:9 * NR]

    c = lax.axis_index("c")
    s = lax.axis_index("s")
    wid = c * NS + s

    def off_of(k):
        return jnp.minimum((k * NW + wid) * CE, E - CE)

    def pf_rc(m, k):
        pltpu.async_copy(adj_hbm.at[pl.ds(0, 2), pl.ds(off_of(k), CE)],
                         rcbufs[m], rcsems[m])

    def pf_vals(m, k):
        pltpu.async_copy(vals_hbm.at[pl.ds(off_of(k), CE)],
                         vbufs[m], vsems[m])

    def wait_rc(m):
        pltpu.make_async_copy(adj_hbm.at[pl.ds(0, 2), pl.ds(0, CE)],
                              rcbufs[m], rcsems[m]).wait()

    def wait_vals(m):
        pltpu.make_async_copy(vals_hbm.at[pl.ds(0, CE)],
                              vbufs[m], vsems[m]).wait()

    def issue_gather(a, m, h):
        pltpu.async_copy(hw_hbm.at[rcbufs[m].at[1, pl.ds(h * SUB, SUB)]],
                         gbufs[a], gsems[a])

    def wait_gather(a):
        pltpu.make_async_copy(hw_hbm.at[rcbufs[0].at[1, pl.ds(0, SUB)]],
                              gbufs[a], gsems[a]).wait()

    def issue_scatter(a):
        pltpu.async_copy(sbufs[a], acc.at[rbufs[a]], ssems[a], add=True)

    def wait_scatter(a):
        pltpu.make_async_copy(sbufs[a], acc.at[rbufs[a]], ssems[a]).wait()

    def copy_rows(a, m, h):
        for t in range(SUB // LANES):
            rbufs[a][pl.ds(LANES * t, LANES)] = (
                rcbufs[m][0, pl.ds(h * SUB + LANES * t, LANES)])

    def scale_rows(a, m, h):
        @plsc.parallel_loop(0, SUB, step=1, unroll=8)
        def _(e):
            v = plsc.load_gather(vbufs[m],
                                 [lax.broadcast(h * SUB + e, (LANES,))])
            for f in range(D // LANES):
                sl = pl.ds(LANES * f, LANES)
                sbufs[a][e, sl] = gbufs[a][e, sl] * v

    for m in range(NR):
        pf_rc(m, m)
        pf_vals(m, m)

    zeros16 = jnp.zeros((LANES,), jnp.float32)

    @plsc.parallel_loop(0, SUB, step=1, unroll=4)
    def _(e):
        for f in range(D // LANES):
            sbufs[0][e, pl.ds(LANES * f, LANES)] = zeros16

    def copy_zero(r0, nr):
        nfull = nr // SUB
        for j in range(nfull):
            pltpu.sync_copy(sbufs[0], acc.at[pl.ds(r0 + j * SUB, SUB)])
        rem = nr - nfull * SUB
        if rem:
            pltpu.sync_copy(sbufs[0].at[pl.ds(0, rem)],
                            acc.at[pl.ds(r0 + nfull * SUB, rem)])

    @pl.when(s < NS - 1)
    def _():
        copy_zero(s * RPT, RPT)

    @pl.when(s == NS - 1)
    def _():
        copy_zero((NS - 1) * RPT, RPT_LAST)

    plsc.subcore_barrier()

    wait_rc(0)
    issue_gather(0, 0, 0)
    issue_gather(1, 0, 1)
    wait_rc(1)
    issue_gather(2, 1, 0)

    def group(g, carry):
        for j in range(6):
            a = j % 3
            m = j // 2
            h = j % 2
            wait_gather(a)
            if j < 3:
                @pl.when(g > 0)
                def _(a=a):
                    wait_scatter(a)
            else:
                wait_scatter(a)
            copy_rows(a, m, h)
            if h == 0:
                wait_vals(m)
            scale_rows(a, m, h)
            issue_scatter(a)
            if h == 1:
                wait_rc(NXT[j][0])
            issue_gather(a, NXT[j][0], NXT[j][1])
            if h == 1:
                pf_rc(m, 3 * g + m + 3)
                pf_vals(m, 3 * g + m + 3)
        return carry

    lax.fori_loop(0, NG, group, 0)

    @pl.when(wid < NEXTRA)
    def _():
        wait_gather(0)
        wait_scatter(0)
        copy_rows(0, 0, 0)
        wait_vals(0)
        scale_rows(0, 0, 0)
        issue_scatter(0)
        wait_gather(1)
        wait_scatter(1)
        copy_rows(1, 0, 1)
        scale_rows(1, 0, 1)
        issue_scatter(1)
        wait_gather(2)
        wait_scatter(2)
        wait_scatter(0)
        wait_scatter(1)
        wait_rc(2)
        wait_vals(1)
        wait_vals(2)

    @pl.when(wid >= NEXTRA)
    def _():
        for a in range(NR):
            wait_gather(a)
            wait_scatter(a)
        wait_rc(2)
        for m in range(NR):
            wait_vals(m)

    plsc.subcore_barrier()

    def write_out(r0, nr):
        @pl.when(c == 0)
        def _():
            pltpu.sync_copy(acc.at[pl.ds(r0, nr)],
                            part0_hbm.at[pl.ds(r0, nr)])

        @pl.when(c == 1)
        def _():
            pltpu.sync_copy(acc.at[pl.ds(r0, nr)],
                            part1_hbm.at[pl.ds(r0, nr)])

    @pl.when(s < NS - 1)
    def _():
        write_out(s * RPT, RPT)

    @pl.when(s == NS - 1)
    def _():
        write_out((NS - 1) * RPT, RPT_LAST)


def _spmm(HW, adj_indices, vals):
    mesh = plsc.VectorSubcoreMesh(core_axis_name="c", subcore_axis_name="s")
    f = functools.partial(
        pl.kernel,
        out_type=(jax.ShapeDtypeStruct((N, D), jnp.float32),
                  jax.ShapeDtypeStruct((N, D), jnp.float32)),
        mesh=mesh,
        compiler_params=pltpu.CompilerParams(needs_layout_passes=False),
        scratch_types=(
            [pltpu.VMEM_SHARED((N, D), jnp.float32)]
            + [pltpu.VMEM((2, CE), jnp.int32)] * NR
            + [pltpu.VMEM((CE,), jnp.float32)] * NR
            + [pltpu.VMEM((SUB,), jnp.int32)] * NR
            + [pltpu.VMEM((SUB, D), jnp.float32)] * NR
            + [pltpu.VMEM((SUB, D), jnp.float32)] * NR
            + [pltpu.SemaphoreType.DMA] * (4 * NR)
        ),
    )(_spmm_body)
    return f(HW, adj_indices, vals)


def _comb_body(p0_ref, p1_ref, b_ref, o_ref):
    o_ref[...] = p0_ref[...] + p1_ref[...] + b_ref[...]


def _combine(p0, p1, b2d):
    bm = 2000
    return pl.pallas_call(
        _comb_body,
        grid=(N // bm,),
        in_specs=[
            pl.BlockSpec((bm, D), lambda i: (i, 0)),
            pl.BlockSpec((bm, D), lambda i: (i, 0)),
            pl.BlockSpec((1, D), lambda i: (0, 0)),
        ],
        out_specs=pl.BlockSpec((bm, D), lambda i: (i, 0)),
        out_shape=jax.ShapeDtypeStruct((N, D), jnp.float32),
    )(p0, p1, b2d)


def kernel(X, adj_indices, adj_values, W, b, mediators):
    HW = _matmul(X, W)
    p0, p1 = _spmm(HW, adj_indices, adj_values)
    return _combine(p0, p1, b.reshape(1, D))
